# baseline (speedup 1.0000x reference)
import functools

import jax
import jax.numpy as jnp
from jax import lax
from jax.experimental import pallas as pl
from jax.experimental.pallas import tpu as pltpu
from jax.experimental.pallas import tpu_sc as plsc

B, L, D = 16, 4096, 1024
NW = 32
G = 8
NG = (B * L) // G
GPB = L // G
NSEG = 3
CB = 4

_mesh = plsc.VectorSubcoreMesh(core_axis_name="c", subcore_axis_name="s")


@functools.partial(
    pl.kernel,
    mesh=_mesh,
    out_type=jax.ShapeDtypeStruct((NG, G, D), jnp.float32),
    scratch_types=[
        pltpu.VMEM((NSEG * NW, 16), jnp.int32),
        pltpu.VMEM((CB, G, D), jnp.float32),
        pltpu.VMEM((CB, G, D), jnp.float32),
        pltpu.VMEM((CB, G, D), jnp.float32),
        pltpu.VMEM((G, D), jnp.float32),
        pltpu.SemaphoreType.DMA,
        pltpu.SemaphoreType.DMA,
        pltpu.SemaphoreType.DMA,
        pltpu.SemaphoreType.DMA,
        pltpu.SemaphoreType.DMA,
    ],
)
def _squeeze_sc(x_hbm, nv_hbm, z_hbm, out_hbm,
                nv_v, cb0, cb1, zbuf, bbuf, is0, is1, os0, os1, zsem):
    wid = lax.axis_index("s") * 2 + lax.axis_index("c")
    pltpu.async_copy(z_hbm, zbuf, zsem)
    pltpu.sync_copy(nv_hbm, nv_v)
    pltpu.make_async_copy(z_hbm, zbuf, zsem).wait()
    bufs = ((cb0, is0, os0), (cb1, is1, os1))

    def _segment(it, carry0):
        row = nv_v[it * NW + wid]
        nv = row[0]
        base = row[1]
        end = row[2]
        nfg = nv >> 3
        r = nv & 7

        nch = nfg >> 2
        npairs = nch >> 1

        def _ring(j, carry):
            for b in range(2):
                i = j * 2 + b
                cb, isem, osem = bufs[b]
                pos = base + i * CB

                @pl.when(j >= 1)
                def _drain_prev(cb=cb, osem=osem, pos=pos):
                    pltpu.make_async_copy(
                        cb, out_hbm.at[pl.ds(pos - 2 * CB, CB)], osem
                    ).wait()

                pltpu.async_copy(x_hbm.at[pl.ds(pos, CB)], cb, isem).wait()
                pltpu.async_copy(cb, out_hbm.at[pl.ds(pos, CB)], osem)
            return carry

        lax.fori_loop(0, npairs, _ring, 0)

        @pl.when(npairs >= 1)
        def _drain_ring():
            for b in range(2):
                cb, isem, osem = bufs[b]
                pos = base + (npairs * 2 - 2 + b) * CB
                pltpu.make_async_copy(cb, out_hbm.at[pl.ds(pos, CB)], osem).wait()

        @pl.when((nch & 1) == 1)
        def _odd_chunk():
            pos = base + (nch - 1) * CB
            pltpu.async_copy(x_hbm.at[pl.ds(pos, CB)], cb0, is0).wait()
            pltpu.async_copy(cb0, out_hbm.at[pl.ds(pos, CB)], os0).wait()

        for k in (1, 0):
            size = 1 << k
            pos = base + ((nfg >> (k + 1)) << (k + 1))

            @pl.when((nfg & size) != 0)
            def _rem_copy(pos=pos, size=size):
                pltpu.async_copy(
                    x_hbm.at[pl.ds(pos, size)], cb0.at[pl.ds(0, size)], is0
                ).wait()
                pltpu.async_copy(
                    cb0.at[pl.ds(0, size)], out_hbm.at[pl.ds(pos, size)], os0
                ).wait()

        gb = base + nfg

        @pl.when(r != 0)
        def _boundary():
            pltpu.async_copy(x_hbm.at[gb], bbuf, is0).wait()
            zv = jnp.zeros((16,), jnp.float32)
            for row8 in range(1, G):

                @pl.when(row8 >= r)
                def _zero_row(row8=row8):
                    def _st(c, carry):
                        bbuf[row8, pl.ds(c * 16, 16)] = zv
                        return carry

                    lax.fori_loop(0, D // 16, _st, 0)

            pltpu.async_copy(bbuf, out_hbm.at[gb], os0).wait()

        zstart = gb + (r != 0).astype(jnp.int32)
        mg = end - zstart
        nzc = mg >> 2

        def _zero_chunk(i, carry):
            @pl.when(i >= 4)
            def _drain(i=i):
                pltpu.make_async_copy(
                    zbuf, out_hbm.at[pl.ds(zstart + (i - 4) * CB, CB)], zsem
                ).wait()

            pltpu.async_copy(zbuf, out_hbm.at[pl.ds(zstart + i * CB, CB)], zsem)
            return carry

        lax.fori_loop(0, nzc, _zero_chunk, 0)
        for t in range(4):

            @pl.when(nzc > t)
            def _drain_tail(t=t):
                pltpu.make_async_copy(
                    zbuf, out_hbm.at[pl.ds(zstart + (nzc - 1 - t) * CB, CB)], zsem
                ).wait()

        for k in (1, 0):
            size = 1 << k
            zpos = zstart + ((mg >> (k + 1)) << (k + 1))

            @pl.when((mg & size) != 0)
            def _zero_rem(zpos=zpos, size=size):
                pltpu.async_copy(
                    zbuf.at[pl.ds(0, size)], out_hbm.at[pl.ds(zpos, size)], zsem
                ).wait()

        return carry0

    lax.fori_loop(0, NSEG, _segment, 0)


def kernel(x, x_len):
    xl = x_len.astype(jnp.int32)
    u = L + xl
    U = jnp.sum(u)
    cumu = jnp.cumsum(u)
    t = (U * jnp.arange(NW + 1, dtype=jnp.int32)) // NW
    tb = jnp.clip(jnp.searchsorted(cumu, t, side="right"), 0, B - 1)
    tin = t - (cumu[tb] - u[tb])
    vbt = xl[tb]
    rin = jnp.where(tin <= 2 * vbt, tin // 2, tin - vbt)
    gbound = tb * GPB + (rin + G // 2) // G
    glo = gbound[:-1]
    ghi = gbound[1:]
    b0 = glo // GPB
    rows = []
    for s in range(NSEG):
        bseg = b0 + s
        bidx = jnp.clip(bseg, 0, B - 1)
        lo0 = jnp.maximum(glo, bseg * GPB)
        hi0 = jnp.minimum(ghi, (bseg + 1) * GPB)
        empty = hi0 <= lo0
        lo = jnp.where(empty, glo, lo0)
        hi = jnp.where(empty, glo, hi0)
        nv = jnp.clip(xl[bidx] - (lo - bidx * GPB) * G, 0, (hi - lo) * G)
        rows.append(jnp.stack([nv, lo, hi], axis=1))
    table = jnp.concatenate(rows, axis=0)
    table = jnp.pad(table, ((0, 0), (0, 16 - table.shape[1])))
    zsrc = jnp.zeros((CB, G, D), jnp.float32)
    out = _squeeze_sc(x.reshape(NG, G, D), table, zsrc)
    return out.reshape(B, L, D)
